# baseline (speedup 1.0000x reference)
import functools

import jax
import jax.numpy as jnp
from jax import lax
from jax.experimental import pallas as pl
from jax.experimental.pallas import tpu as pltpu
from jax.experimental.pallas import tpu_sc as plsc

N = 10000
E = 320000
D = 128
HD = 64
BLK = 1000

NC = 2
NS = 16
NW = NC * NS

C = 80
PER_W = E // NW
NCHUNK = PER_W // C
NP = 10240
ROWS_PER_TILE = NP // NS


def _make_sc_spmm(dk, tc_tiling):
    mesh = plsc.VectorSubcoreMesh(core_axis_name="c", subcore_axis_name="s")

    @functools.partial(
        pl.kernel,
        out_type=jax.ShapeDtypeStruct((NC, NP, dk), jnp.float32),
        mesh=mesh,
        compiler_params=pltpu.CompilerParams(use_tc_tiling_on_sc=tc_tiling),
        scratch_types=[
            pltpu.VMEM((C,), jnp.int32),
            pltpu.VMEM((C,), jnp.int32),
            pltpu.VMEM((C,), jnp.int32),
            pltpu.VMEM((C,), jnp.int32),
            pltpu.VMEM((C,), jnp.int32),
            pltpu.VMEM((C,), jnp.int32),
            pltpu.VMEM((C,), jnp.int32),
            pltpu.VMEM((C,), jnp.int32),
            pltpu.VMEM((C,), jnp.float32),
            pltpu.VMEM((C,), jnp.float32),
            pltpu.VMEM((C, dk), jnp.float32),
            pltpu.VMEM((C, dk), jnp.float32),
            pltpu.VMEM((C, dk), jnp.float32),
            pltpu.VMEM((C, dk), jnp.float32),
            pltpu.VMEM_SHARED((NP, dk), jnp.float32),
            pltpu.SemaphoreType.DMA,
            pltpu.SemaphoreType.DMA,
            pltpu.SemaphoreType.DMA,
            pltpu.SemaphoreType.DMA,
            pltpu.SemaphoreType.DMA,
            pltpu.SemaphoreType.DMA,
            pltpu.SemaphoreType.DMA,
            pltpu.SemaphoreType.DMA,
            pltpu.SemaphoreType.DMA,
            pltpu.SemaphoreType.DMA,
            pltpu.SemaphoreType.DMA,
            pltpu.SemaphoreType.DMA,
            pltpu.SemaphoreType.DMA,
            pltpu.SemaphoreType.DMA,
        ],
    )
    def spmm(src_hbm, dst_hbm, w_hbm, x_hbm, out_hbm,
             sq0, sq1, sq2, sq3, dq0, dq1, dq2, dq3, wq0, wq1,
             rows0, rows1, sout0, sout1, acc_sh,
             semg0, semg1, sems0, sems1,
             semsw0, semsw1, semsw2, semsw3,
             semd0, semd1, semd2, semd3, semw0, semw1):
        c = lax.axis_index("c")
        s = lax.axis_index("s")
        wid = s * NC + c
        ebase = wid * PER_W

        srcq = (sq0, sq1, sq2, sq3)
        dstq = (dq0, dq1, dq2, dq3)
        wqs = (wq0, wq1)
        rows = (rows0, rows1)
        souts = (sout0, sout1)
        semg = (semg0, semg1)
        sems = (sems0, sems1)
        semsw = (semsw0, semsw1, semsw2, semsw3)
        semd = (semd0, semd1, semd2, semd3)
        semw = (semw0, semw1)

        def sw_dma(k, q):
            pltpu.async_copy(
                src_hbm.at[pl.ds(ebase + k * C, C)], srcq[q], semsw[q])

        def sw_wait(k, q):
            pltpu.make_async_copy(
                src_hbm.at[pl.ds(ebase + k * C, C)], srcq[q], semsw[q]).wait()

        def dst_dma(k, q):
            pltpu.async_copy(
                dst_hbm.at[pl.ds(ebase + k * C, C)], dstq[q], semd[q])

        def dst_wait(k, q):
            pltpu.make_async_copy(
                dst_hbm.at[pl.ds(ebase + k * C, C)], dstq[q], semd[q]).wait()

        def w_dma(k, b):
            pltpu.async_copy(
                w_hbm.at[pl.ds(ebase + k * C, C)], wqs[b], semw[b])

        def w_wait(k, b):
            pltpu.make_async_copy(
                w_hbm.at[pl.ds(ebase + k * C, C)], wqs[b], semw[b]).wait()

        def gather_start(b, q):
            pltpu.async_copy(x_hbm.at[srcq[q]], rows[b], semg[b])

        def gather_wait(b, q):
            pltpu.make_async_copy(x_hbm.at[srcq[q]], rows[b], semg[b]).wait()

        def scatter_start(b, q):
            pltpu.async_copy(souts[b], acc_sh.at[dstq[q]], sems[b], add=True)

        def scatter_wait(b, q):
            pltpu.make_async_copy(souts[b], acc_sh.at[dstq[q]], sems[b]).wait()

        sw_dma(0, 0)
        sw_dma(1, 1)
        sw_dma(2, 2)
        dst_dma(0, 0)
        w_dma(0, 0)
        dst_dma(1, 1)
        w_dma(1, 1)

        zeros16 = jnp.zeros((16,), jnp.float32)

        def zfill(i, carry):
            for j in range(dk // 16):
                sout0[i, pl.ds(j * 16, 16)] = zeros16
            return carry

        lax.fori_loop(0, C, zfill, 0)
        for r in range(ROWS_PER_TILE // C):
            pltpu.sync_copy(
                sout0, acc_sh.at[pl.ds(s * ROWS_PER_TILE + r * C, C)])
        plsc.subcore_barrier()

        sw_wait(0, 0)
        gather_start(0, 0)
        sw_wait(1, 1)
        gather_start(1, 1)

        def step(k, b, q):
            q2 = (q + 2) % 4
            q3 = (q + 3) % 4
            gather_wait(b, q)
            dst_wait(k, q)
            w_wait(k, b)

            @pl.when(k >= 2)
            def _():
                scatter_wait(b, q)

            for t in range(C // 16):
                wv = wqs[b][pl.ds(t * 16, 16)]
                for e in range(16):
                    r = t * 16 + e
                    wbc = jnp.full((16,), wv[e], jnp.float32)
                    for j in range(dk // 16):
                        souts[b][r, pl.ds(j * 16, 16)] = (
                            rows[b][r, pl.ds(j * 16, 16)] * wbc)
            scatter_start(b, q)

            @pl.when(k + 3 < NCHUNK)
            def _():
                sw_dma(k + 3, q3)

            @pl.when(k + 2 < NCHUNK)
            def _():
                sw_wait(k + 2, q2)
                gather_start(b, q2)
                dst_dma(k + 2, q2)
                w_dma(k + 2, b)

        def outer(k4, carry):
            for u in range(4):
                step(k4 * 4 + u, u % 2, u)
            return carry

        lax.fori_loop(0, NCHUNK // 4, outer, 0)
        for u in range(NCHUNK % 4):
            step(NCHUNK - (NCHUNK % 4) + u, u % 2, u)
        scatter_wait(0, 0)
        scatter_wait(1, 1)
        plsc.subcore_barrier()

        base_r = s * ROWS_PER_TILE
        pltpu.sync_copy(acc_sh.at[pl.ds(base_r, ROWS_PER_TILE)],
                        out_hbm.at[c, pl.ds(base_r, ROWS_PER_TILE)])

    return spmm


_sc_spmm_full = _make_sc_spmm(D, True)
_sc_spmm_half = _make_sc_spmm(HD, False)


def _stage_a_body(g0_ref, g1_ref, w0_ref, g_ref, u_ref):
    g = g0_ref[0] + g1_ref[0]
    g_ref[...] = g
    u_ref[...] = jnp.dot(g, w0_ref[...], preferred_element_type=jnp.float32)


def _stage_a(gp, W0):
    return pl.pallas_call(
        _stage_a_body,
        grid=(N // BLK,),
        in_specs=[
            pl.BlockSpec((1, BLK, D), lambda i: (0, i, 0)),
            pl.BlockSpec((1, BLK, D), lambda i: (1, i, 0)),
            pl.BlockSpec((D, HD), lambda i: (0, 0)),
        ],
        out_specs=[
            pl.BlockSpec((BLK, D), lambda i: (i, 0)),
            pl.BlockSpec((BLK, HD), lambda i: (i, 0)),
        ],
        out_shape=[
            jax.ShapeDtypeStruct((N, D), jnp.float32),
            jax.ShapeDtypeStruct((N, HD), jnp.float32),
        ],
    )(gp, gp, W0)


def _softmax(x):
    m = jnp.max(x, axis=-1, keepdims=True)
    e = jnp.exp(x - m)
    return e / jnp.sum(e, axis=-1, keepdims=True)


def _stage_b_body(a0_ref, a1_ref, b_ref, w1_ref, h1_ref, v_ref):
    h1 = _softmax(a0_ref[0] + a1_ref[0] + b_ref[...])
    h1_ref[...] = h1
    v_ref[...] = jnp.dot(h1, w1_ref[...], preferred_element_type=jnp.float32)


def _stage_b(ap, b0, W1):
    return pl.pallas_call(
        _stage_b_body,
        grid=(N // BLK,),
        in_specs=[
            pl.BlockSpec((1, BLK, HD), lambda i: (0, i, 0)),
            pl.BlockSpec((1, BLK, HD), lambda i: (1, i, 0)),
            pl.BlockSpec((1, HD), lambda i: (0, 0)),
            pl.BlockSpec((HD, HD), lambda i: (0, 0)),
        ],
        out_specs=[
            pl.BlockSpec((BLK, HD), lambda i: (i, 0)),
            pl.BlockSpec((BLK, HD), lambda i: (i, 0)),
        ],
        out_shape=[
            jax.ShapeDtypeStruct((N, HD), jnp.float32),
            jax.ShapeDtypeStruct((N, HD), jnp.float32),
        ],
    )(ap, ap, b0.reshape(1, HD), W1)


def _stage_c_body(c0_ref, c1_ref, b_ref, h1_ref, g_ref, out_ref):
    h2 = _softmax(c0_ref[0] + c1_ref[0] + b_ref[...])
    cat = jnp.concatenate([h1_ref[...], h2], axis=1)
    out_ref[...] = jnp.tanh(cat + g_ref[...])


def _stage_c(cp, b1, h1, g):
    return pl.pallas_call(
        _stage_c_body,
        grid=(N // BLK,),
        in_specs=[
            pl.BlockSpec((1, BLK, HD), lambda i: (0, i, 0)),
            pl.BlockSpec((1, BLK, HD), lambda i: (1, i, 0)),
            pl.BlockSpec((1, HD), lambda i: (0, 0)),
            pl.BlockSpec((BLK, HD), lambda i: (i, 0)),
            pl.BlockSpec((BLK, D), lambda i: (i, 0)),
        ],
        out_specs=pl.BlockSpec((BLK, D), lambda i: (i, 0)),
        out_shape=jax.ShapeDtypeStruct((N, D), jnp.float32),
    )(cp, cp, b1.reshape(1, HD), h1, g)


def kernel(edge_index, edge_weight, inputs_emb, W0, b0, W1, b1):
    dst = edge_index[0]
    src = edge_index[1]
    gp = _sc_spmm_full(src, dst, edge_weight, inputs_emb)
    g, u = _stage_a(gp, W0)
    ap = _sc_spmm_half(src, dst, edge_weight, u)
    h1, v = _stage_b(ap, b0, W1)
    cp = _sc_spmm_half(src, dst, edge_weight, v)
    return _stage_c(cp, b1, h1, g)


# speedup vs baseline: 1.0046x; 1.0046x over previous
import functools

import jax
import jax.numpy as jnp
from jax import lax
from jax.experimental import pallas as pl
from jax.experimental.pallas import tpu as pltpu
from jax.experimental.pallas import tpu_sc as plsc

N = 10000
E = 320000
D = 128
HD = 64
BLK = 1000

NC = 2
NS = 16
NW = NC * NS

C = 80
PER_W = E // NW
NCHUNK = PER_W // C
NP = 10240
ROWS_PER_TILE = NP // NS


def _make_sc_spmm(dk, tc_tiling):
    mesh = plsc.VectorSubcoreMesh(core_axis_name="c", subcore_axis_name="s")

    @functools.partial(
        pl.kernel,
        out_type=jax.ShapeDtypeStruct((NC, NP, dk), jnp.float32),
        mesh=mesh,
        compiler_params=pltpu.CompilerParams(use_tc_tiling_on_sc=tc_tiling),
        scratch_types=[
            pltpu.VMEM((C,), jnp.int32),
            pltpu.VMEM((C,), jnp.int32),
            pltpu.VMEM((C,), jnp.int32),
            pltpu.VMEM((C,), jnp.int32),
            pltpu.VMEM((C,), jnp.int32),
            pltpu.VMEM((C,), jnp.int32),
            pltpu.VMEM((C,), jnp.int32),
            pltpu.VMEM((C,), jnp.int32),
            pltpu.VMEM((C,), jnp.float32),
            pltpu.VMEM((C,), jnp.float32),
            pltpu.VMEM((C, dk), jnp.float32),
            pltpu.VMEM((C, dk), jnp.float32),
            pltpu.VMEM((C, dk), jnp.float32),
            pltpu.VMEM((C, dk), jnp.float32),
            pltpu.VMEM_SHARED((NP, dk), jnp.float32),
            pltpu.SemaphoreType.DMA,
            pltpu.SemaphoreType.DMA,
            pltpu.SemaphoreType.DMA,
            pltpu.SemaphoreType.DMA,
            pltpu.SemaphoreType.DMA,
            pltpu.SemaphoreType.DMA,
            pltpu.SemaphoreType.DMA,
            pltpu.SemaphoreType.DMA,
            pltpu.SemaphoreType.DMA,
            pltpu.SemaphoreType.DMA,
            pltpu.SemaphoreType.DMA,
            pltpu.SemaphoreType.DMA,
            pltpu.SemaphoreType.DMA,
            pltpu.SemaphoreType.DMA,
        ],
    )
    def spmm(src_hbm, dst_hbm, w_hbm, x_hbm, out_hbm,
             sq0, sq1, sq2, sq3, dq0, dq1, dq2, dq3, wq0, wq1,
             rows0, rows1, sout0, sout1, acc_sh,
             semg0, semg1, sems0, sems1,
             semsw0, semsw1, semsw2, semsw3,
             semd0, semd1, semd2, semd3, semw0, semw1):
        c = lax.axis_index("c")
        s = lax.axis_index("s")
        wid = s * NC + c
        ebase = wid * PER_W

        srcq = (sq0, sq1, sq2, sq3)
        dstq = (dq0, dq1, dq2, dq3)
        wqs = (wq0, wq1)
        rows = (rows0, rows1)
        souts = (sout0, sout1)
        semg = (semg0, semg1)
        sems = (sems0, sems1)
        semsw = (semsw0, semsw1, semsw2, semsw3)
        semd = (semd0, semd1, semd2, semd3)
        semw = (semw0, semw1)

        def sw_dma(k, q):
            pltpu.async_copy(
                src_hbm.at[pl.ds(ebase + k * C, C)], srcq[q], semsw[q])

        def sw_wait(k, q):
            pltpu.make_async_copy(
                src_hbm.at[pl.ds(ebase + k * C, C)], srcq[q], semsw[q]).wait()

        def dst_dma(k, q):
            pltpu.async_copy(
                dst_hbm.at[pl.ds(ebase + k * C, C)], dstq[q], semd[q])

        def dst_wait(k, q):
            pltpu.make_async_copy(
                dst_hbm.at[pl.ds(ebase + k * C, C)], dstq[q], semd[q]).wait()

        def w_dma(k, b):
            pltpu.async_copy(
                w_hbm.at[pl.ds(ebase + k * C, C)], wqs[b], semw[b])

        def w_wait(k, b):
            pltpu.make_async_copy(
                w_hbm.at[pl.ds(ebase + k * C, C)], wqs[b], semw[b]).wait()

        def gather_start(b, q):
            pltpu.async_copy(x_hbm.at[srcq[q]], rows[b], semg[b])

        def gather_wait(b, q):
            pltpu.make_async_copy(x_hbm.at[srcq[q]], rows[b], semg[b]).wait()

        def scatter_start(b, q):
            pltpu.async_copy(souts[b], acc_sh.at[dstq[q]], sems[b], add=True)

        def scatter_wait(b, q):
            pltpu.make_async_copy(souts[b], acc_sh.at[dstq[q]], sems[b]).wait()

        sw_dma(0, 0)
        sw_dma(1, 1)
        sw_dma(2, 2)
        dst_dma(0, 0)
        w_dma(0, 0)
        dst_dma(1, 1)
        w_dma(1, 1)

        zeros16 = jnp.zeros((16,), jnp.float32)

        def zfill(i, carry):
            for j in range(dk // 16):
                sout0[i, pl.ds(j * 16, 16)] = zeros16
            return carry

        lax.fori_loop(0, C, zfill, 0)
        for r in range(ROWS_PER_TILE // C):
            pltpu.sync_copy(
                sout0, acc_sh.at[pl.ds(s * ROWS_PER_TILE + r * C, C)])
        plsc.subcore_barrier()

        sw_wait(0, 0)
        gather_start(0, 0)
        sw_wait(1, 1)
        gather_start(1, 1)

        def step(k, b, q):
            q2 = (q + 2) % 4
            q3 = (q + 3) % 4
            gather_wait(b, q)
            dst_wait(k, q)
            w_wait(k, b)

            @pl.when(k >= 2)
            def _():
                scatter_wait(b, q)

            for t in range(C // 16):
                wv = wqs[b][pl.ds(t * 16, 16)]
                wbcs = [jnp.full((16,), wv[e], jnp.float32) for e in range(16)]
                for e in range(16):
                    r = t * 16 + e
                    for j in range(dk // 16):
                        souts[b][r, pl.ds(j * 16, 16)] = (
                            rows[b][r, pl.ds(j * 16, 16)] * wbcs[e])
            scatter_start(b, q)

            @pl.when(k + 3 < NCHUNK)
            def _():
                sw_dma(k + 3, q3)

            @pl.when(k + 2 < NCHUNK)
            def _():
                sw_wait(k + 2, q2)
                gather_start(b, q2)
                dst_dma(k + 2, q2)
                w_dma(k + 2, b)

        def outer(k4, carry):
            for u in range(4):
                step(k4 * 4 + u, u % 2, u)
            return carry

        lax.fori_loop(0, NCHUNK // 4, outer, 0)
        for u in range(NCHUNK % 4):
            step(NCHUNK - (NCHUNK % 4) + u, u % 2, u)
        scatter_wait(0, 0)
        scatter_wait(1, 1)
        plsc.subcore_barrier()

        base_r = s * ROWS_PER_TILE
        pltpu.sync_copy(acc_sh.at[pl.ds(base_r, ROWS_PER_TILE)],
                        out_hbm.at[c, pl.ds(base_r, ROWS_PER_TILE)])

    return spmm


_sc_spmm_full = _make_sc_spmm(D, True)
_sc_spmm_half = _make_sc_spmm(HD, False)


def _stage_a_body(g0_ref, g1_ref, w0_ref, g_ref, u_ref):
    g = g0_ref[0] + g1_ref[0]
    g_ref[...] = g
    u_ref[...] = jnp.dot(g, w0_ref[...], preferred_element_type=jnp.float32)


def _stage_a(gp, W0):
    return pl.pallas_call(
        _stage_a_body,
        grid=(N // BLK,),
        in_specs=[
            pl.BlockSpec((1, BLK, D), lambda i: (0, i, 0)),
            pl.BlockSpec((1, BLK, D), lambda i: (1, i, 0)),
            pl.BlockSpec((D, HD), lambda i: (0, 0)),
        ],
        out_specs=[
            pl.BlockSpec((BLK, D), lambda i: (i, 0)),
            pl.BlockSpec((BLK, HD), lambda i: (i, 0)),
        ],
        out_shape=[
            jax.ShapeDtypeStruct((N, D), jnp.float32),
            jax.ShapeDtypeStruct((N, HD), jnp.float32),
        ],
    )(gp, gp, W0)


def _softmax(x):
    m = jnp.max(x, axis=-1, keepdims=True)
    e = jnp.exp(x - m)
    return e / jnp.sum(e, axis=-1, keepdims=True)


def _stage_b_body(a0_ref, a1_ref, b_ref, w1_ref, h1_ref, v_ref):
    h1 = _softmax(a0_ref[0] + a1_ref[0] + b_ref[...])
    h1_ref[...] = h1
    v_ref[...] = jnp.dot(h1, w1_ref[...], preferred_element_type=jnp.float32)


def _stage_b(ap, b0, W1):
    return pl.pallas_call(
        _stage_b_body,
        grid=(N // BLK,),
        in_specs=[
            pl.BlockSpec((1, BLK, HD), lambda i: (0, i, 0)),
            pl.BlockSpec((1, BLK, HD), lambda i: (1, i, 0)),
            pl.BlockSpec((1, HD), lambda i: (0, 0)),
            pl.BlockSpec((HD, HD), lambda i: (0, 0)),
        ],
        out_specs=[
            pl.BlockSpec((BLK, HD), lambda i: (i, 0)),
            pl.BlockSpec((BLK, HD), lambda i: (i, 0)),
        ],
        out_shape=[
            jax.ShapeDtypeStruct((N, HD), jnp.float32),
            jax.ShapeDtypeStruct((N, HD), jnp.float32),
        ],
    )(ap, ap, b0.reshape(1, HD), W1)


def _stage_c_body(c0_ref, c1_ref, b_ref, h1_ref, g_ref, out_ref):
    h2 = _softmax(c0_ref[0] + c1_ref[0] + b_ref[...])
    cat = jnp.concatenate([h1_ref[...], h2], axis=1)
    out_ref[...] = jnp.tanh(cat + g_ref[...])


def _stage_c(cp, b1, h1, g):
    return pl.pallas_call(
        _stage_c_body,
        grid=(N // BLK,),
        in_specs=[
            pl.BlockSpec((1, BLK, HD), lambda i: (0, i, 0)),
            pl.BlockSpec((1, BLK, HD), lambda i: (1, i, 0)),
            pl.BlockSpec((1, HD), lambda i: (0, 0)),
            pl.BlockSpec((BLK, HD), lambda i: (i, 0)),
            pl.BlockSpec((BLK, D), lambda i: (i, 0)),
        ],
        out_specs=pl.BlockSpec((BLK, D), lambda i: (i, 0)),
        out_shape=jax.ShapeDtypeStruct((N, D), jnp.float32),
    )(cp, cp, b1.reshape(1, HD), h1, g)


def kernel(edge_index, edge_weight, inputs_emb, W0, b0, W1, b1):
    dst = edge_index[0]
    src = edge_index[1]
    gp = _sc_spmm_full(src, dst, edge_weight, inputs_emb)
    g, u = _stage_a(gp, W0)
    ap = _sc_spmm_half(src, dst, edge_weight, u)
    h1, v = _stage_b(ap, b0, W1)
    cp = _sc_spmm_half(src, dst, edge_weight, v)
    return _stage_c(cp, b1, h1, g)


# speedup vs baseline: 1.0083x; 1.0037x over previous
import functools

import jax
import jax.numpy as jnp
from jax import lax
from jax.experimental import pallas as pl
from jax.experimental.pallas import tpu as pltpu
from jax.experimental.pallas import tpu_sc as plsc

N = 10000
E = 320000
D = 128
HD = 64
BLK = 1000

NC = 2
NS = 16
NW = NC * NS

C = 80
PER_W = E // NW
NCHUNK = PER_W // C
NP = 10240
ROWS_PER_TILE = NP // NS


def _make_sc_spmm(dk, tc_tiling):
    mesh = plsc.VectorSubcoreMesh(core_axis_name="c", subcore_axis_name="s")

    @functools.partial(
        pl.kernel,
        out_type=jax.ShapeDtypeStruct((NC, NP, dk), jnp.float32),
        mesh=mesh,
        compiler_params=pltpu.CompilerParams(use_tc_tiling_on_sc=tc_tiling),
        scratch_types=[
            pltpu.VMEM((C,), jnp.int32),
            pltpu.VMEM((C,), jnp.int32),
            pltpu.VMEM((C,), jnp.int32),
            pltpu.VMEM((C,), jnp.int32),
            pltpu.VMEM((C,), jnp.int32),
            pltpu.VMEM((C,), jnp.int32),
            pltpu.VMEM((C,), jnp.int32),
            pltpu.VMEM((C,), jnp.int32),
            pltpu.VMEM((C,), jnp.float32),
            pltpu.VMEM((C,), jnp.float32),
            pltpu.VMEM((C, dk), jnp.float32),
            pltpu.VMEM((C, dk), jnp.float32),
            pltpu.VMEM((C, dk), jnp.float32),
            pltpu.VMEM((C, dk), jnp.float32),
            pltpu.VMEM_SHARED((NP, dk), jnp.float32),
            pltpu.SemaphoreType.DMA,
            pltpu.SemaphoreType.DMA,
            pltpu.SemaphoreType.DMA,
            pltpu.SemaphoreType.DMA,
            pltpu.SemaphoreType.DMA,
            pltpu.SemaphoreType.DMA,
            pltpu.SemaphoreType.DMA,
            pltpu.SemaphoreType.DMA,
            pltpu.SemaphoreType.DMA,
            pltpu.SemaphoreType.DMA,
            pltpu.SemaphoreType.DMA,
            pltpu.SemaphoreType.DMA,
            pltpu.SemaphoreType.DMA,
            pltpu.SemaphoreType.DMA,
        ],
    )
    def spmm(src_hbm, dst_hbm, w_hbm, x_hbm, out_hbm,
             sq0, sq1, sq2, sq3, dq0, dq1, dq2, dq3, wq0, wq1,
             rows0, rows1, sout0, sout1, acc_sh,
             semg0, semg1, sems0, sems1,
             semsw0, semsw1, semsw2, semsw3,
             semd0, semd1, semd2, semd3, semw0, semw1):
        c = lax.axis_index("c")
        s = lax.axis_index("s")
        wid = s * NC + c
        ebase = wid * PER_W

        srcq = (sq0, sq1, sq2, sq3)
        dstq = (dq0, dq1, dq2, dq3)
        wqs = (wq0, wq1)
        rows = (rows0, rows1)
        souts = (sout0, sout1)
        semg = (semg0, semg1)
        sems = (sems0, sems1)
        semsw = (semsw0, semsw1, semsw2, semsw3)
        semd = (semd0, semd1, semd2, semd3)
        semw = (semw0, semw1)

        def sw_dma(k, q):
            pltpu.async_copy(
                src_hbm.at[pl.ds(ebase + k * C, C)], srcq[q], semsw[q])

        def sw_wait(k, q):
            pltpu.make_async_copy(
                src_hbm.at[pl.ds(ebase + k * C, C)], srcq[q], semsw[q]).wait()

        def dst_dma(k, q):
            pltpu.async_copy(
                dst_hbm.at[pl.ds(ebase + k * C, C)], dstq[q], semd[q])

        def dst_wait(k, q):
            pltpu.make_async_copy(
                dst_hbm.at[pl.ds(ebase + k * C, C)], dstq[q], semd[q]).wait()

        def w_dma(k, b):
            pltpu.async_copy(
                w_hbm.at[pl.ds(ebase + k * C, C)], wqs[b], semw[b])

        def w_wait(k, b):
            pltpu.make_async_copy(
                w_hbm.at[pl.ds(ebase + k * C, C)], wqs[b], semw[b]).wait()

        def gather_start(b, q):
            pltpu.async_copy(x_hbm.at[srcq[q]], rows[b], semg[b])

        def gather_wait(b, q):
            pltpu.make_async_copy(x_hbm.at[srcq[q]], rows[b], semg[b]).wait()

        def scatter_start(b, q):
            pltpu.async_copy(souts[b], acc_sh.at[dstq[q]], sems[b], add=True)

        def scatter_wait(b, q):
            pltpu.make_async_copy(souts[b], acc_sh.at[dstq[q]], sems[b]).wait()

        sw_dma(0, 0)
        sw_dma(1, 1)
        sw_dma(2, 2)
        dst_dma(0, 0)
        w_dma(0, 0)
        dst_dma(1, 1)
        w_dma(1, 1)

        zeros16 = jnp.zeros((16,), jnp.float32)

        def zfill(i, carry):
            for j in range(dk // 16):
                sout0[i, pl.ds(j * 16, 16)] = zeros16
            return carry

        lax.fori_loop(0, C, zfill, 0)
        for r in range(ROWS_PER_TILE // C):
            pltpu.async_copy(
                sout0, acc_sh.at[pl.ds(s * ROWS_PER_TILE + r * C, C)],
                semg0)
        for r in range(ROWS_PER_TILE // C):
            pltpu.make_async_copy(
                sout0, acc_sh.at[pl.ds(s * ROWS_PER_TILE + r * C, C)],
                semg0).wait()
        plsc.subcore_barrier()

        sw_wait(0, 0)
        gather_start(0, 0)
        sw_wait(1, 1)
        gather_start(1, 1)

        def step(k, b, q):
            q2 = (q + 2) % 4
            q3 = (q + 3) % 4
            gather_wait(b, q)
            dst_wait(k, q)
            w_wait(k, b)

            @pl.when(k >= 2)
            def _():
                scatter_wait(b, q)

            for t in range(C // 16):
                wv = wqs[b][pl.ds(t * 16, 16)]
                wbcs = [jnp.full((16,), wv[e], jnp.float32) for e in range(16)]
                for e in range(16):
                    r = t * 16 + e
                    for j in range(dk // 16):
                        souts[b][r, pl.ds(j * 16, 16)] = (
                            rows[b][r, pl.ds(j * 16, 16)] * wbcs[e])
            scatter_start(b, q)

            @pl.when(k + 3 < NCHUNK)
            def _():
                sw_dma(k + 3, q3)

            @pl.when(k + 2 < NCHUNK)
            def _():
                sw_wait(k + 2, q2)
                gather_start(b, q2)
                dst_dma(k + 2, q2)
                w_dma(k + 2, b)

        def outer(k4, carry):
            for u in range(4):
                step(k4 * 4 + u, u % 2, u)
            return carry

        lax.fori_loop(0, NCHUNK // 4, outer, 0)
        for u in range(NCHUNK % 4):
            step(NCHUNK - (NCHUNK % 4) + u, u % 2, u)
        scatter_wait(0, 0)
        scatter_wait(1, 1)
        plsc.subcore_barrier()

        base_r = s * ROWS_PER_TILE
        pltpu.sync_copy(acc_sh.at[pl.ds(base_r, ROWS_PER_TILE)],
                        out_hbm.at[c, pl.ds(base_r, ROWS_PER_TILE)])

    return spmm


_sc_spmm_full = _make_sc_spmm(D, True)
_sc_spmm_half = _make_sc_spmm(HD, False)


def _stage_a_body(g0_ref, g1_ref, w0_ref, g_ref, u_ref):
    g = g0_ref[0] + g1_ref[0]
    g_ref[...] = g
    u_ref[...] = jnp.dot(g, w0_ref[...], preferred_element_type=jnp.float32)


def _stage_a(gp, W0):
    return pl.pallas_call(
        _stage_a_body,
        grid=(N // BLK,),
        in_specs=[
            pl.BlockSpec((1, BLK, D), lambda i: (0, i, 0)),
            pl.BlockSpec((1, BLK, D), lambda i: (1, i, 0)),
            pl.BlockSpec((D, HD), lambda i: (0, 0)),
        ],
        out_specs=[
            pl.BlockSpec((BLK, D), lambda i: (i, 0)),
            pl.BlockSpec((BLK, HD), lambda i: (i, 0)),
        ],
        out_shape=[
            jax.ShapeDtypeStruct((N, D), jnp.float32),
            jax.ShapeDtypeStruct((N, HD), jnp.float32),
        ],
    )(gp, gp, W0)


def _softmax(x):
    m = jnp.max(x, axis=-1, keepdims=True)
    e = jnp.exp(x - m)
    return e / jnp.sum(e, axis=-1, keepdims=True)


def _stage_b_body(a0_ref, a1_ref, b_ref, w1_ref, h1_ref, v_ref):
    h1 = _softmax(a0_ref[0] + a1_ref[0] + b_ref[...])
    h1_ref[...] = h1
    v_ref[...] = jnp.dot(h1, w1_ref[...], preferred_element_type=jnp.float32)


def _stage_b(ap, b0, W1):
    return pl.pallas_call(
        _stage_b_body,
        grid=(N // BLK,),
        in_specs=[
            pl.BlockSpec((1, BLK, HD), lambda i: (0, i, 0)),
            pl.BlockSpec((1, BLK, HD), lambda i: (1, i, 0)),
            pl.BlockSpec((1, HD), lambda i: (0, 0)),
            pl.BlockSpec((HD, HD), lambda i: (0, 0)),
        ],
        out_specs=[
            pl.BlockSpec((BLK, HD), lambda i: (i, 0)),
            pl.BlockSpec((BLK, HD), lambda i: (i, 0)),
        ],
        out_shape=[
            jax.ShapeDtypeStruct((N, HD), jnp.float32),
            jax.ShapeDtypeStruct((N, HD), jnp.float32),
        ],
    )(ap, ap, b0.reshape(1, HD), W1)


def _stage_c_body(c0_ref, c1_ref, b_ref, h1_ref, g_ref, out_ref):
    h2 = _softmax(c0_ref[0] + c1_ref[0] + b_ref[...])
    cat = jnp.concatenate([h1_ref[...], h2], axis=1)
    out_ref[...] = jnp.tanh(cat + g_ref[...])


def _stage_c(cp, b1, h1, g):
    return pl.pallas_call(
        _stage_c_body,
        grid=(N // BLK,),
        in_specs=[
            pl.BlockSpec((1, BLK, HD), lambda i: (0, i, 0)),
            pl.BlockSpec((1, BLK, HD), lambda i: (1, i, 0)),
            pl.BlockSpec((1, HD), lambda i: (0, 0)),
            pl.BlockSpec((BLK, HD), lambda i: (i, 0)),
            pl.BlockSpec((BLK, D), lambda i: (i, 0)),
        ],
        out_specs=pl.BlockSpec((BLK, D), lambda i: (i, 0)),
        out_shape=jax.ShapeDtypeStruct((N, D), jnp.float32),
    )(cp, cp, b1.reshape(1, HD), h1, g)


def kernel(edge_index, edge_weight, inputs_emb, W0, b0, W1, b1):
    dst = edge_index[0]
    src = edge_index[1]
    gp = _sc_spmm_full(src, dst, edge_weight, inputs_emb)
    g, u = _stage_a(gp, W0)
    ap = _sc_spmm_half(src, dst, edge_weight, u)
    h1, v = _stage_b(ap, b0, W1)
    cp = _sc_spmm_half(src, dst, edge_weight, v)
    return _stage_c(cp, b1, h1, g)
